# baseline (speedup 1.0000x reference)
import functools

import jax
import jax.numpy as jnp
from jax.experimental import pallas as pl
from jax.experimental.pallas import tpu as pltpu

ROWS = 32
VOCAB = 1_000_000
BLK = 16384
GRID = (VOCAB + BLK - 1) // BLK

_TINY = 1.1754943508222875e-38
_BIG_IDX = 2**30


def _threefry_bits(j):
    rotations = ((13, 15, 26, 6), (17, 29, 16, 24))
    k0 = jnp.uint32(0)
    k1 = jnp.uint32(42)
    ks = (k0, k1, jnp.uint32(0x1BD11BDA) ^ k0 ^ k1)
    x0 = jnp.zeros_like(j) + ks[0]
    x1 = j + ks[1]

    def rotl(x, d):
        return (x << jnp.uint32(d)) | (x >> jnp.uint32(32 - d))

    for i in range(5):
        for r in rotations[i % 2]:
            x0 = x0 + x1
            x1 = rotl(x1, r)
            x1 = x0 ^ x1
        x0 = x0 + ks[(i + 1) % 3]
        x1 = x1 + ks[(i + 2) % 3] + jnp.uint32(i + 1)
    return x0 ^ x1


def _sample_kernel(x_ref, out_ref, best_ref, bestidx_ref):
    b = pl.program_id(0)

    @pl.when(b == 0)
    def _init():
        best_ref[...] = jnp.full((ROWS, 1), -jnp.inf, jnp.float32)
        bestidx_ref[...] = jnp.zeros((ROWS, 1), jnp.int32)

    rows = jax.lax.broadcasted_iota(jnp.uint32, (ROWS, BLK), 0)
    local = jax.lax.broadcasted_iota(jnp.uint32, (ROWS, BLK), 1)
    col = jnp.uint32(b * BLK) + local
    j = rows * jnp.uint32(VOCAB) + col

    bits = _threefry_bits(j)
    ubits = (bits >> jnp.uint32(9)) | jnp.uint32(0x3F800000)
    f = jax.lax.bitcast_convert_type(ubits, jnp.float32) - jnp.float32(1.0)
    tiny = jnp.float32(_TINY)
    u = jnp.maximum(f + tiny, tiny)
    g = -jnp.log(-jnp.log(u))

    score = g + x_ref[...]
    valid = col < jnp.uint32(VOCAB)
    score = jnp.where(valid, score, -jnp.inf)

    m = jnp.max(score, axis=1, keepdims=True)
    gcol = jnp.where(score == m, col.astype(jnp.int32), jnp.int32(_BIG_IDX))
    am = jnp.min(gcol, axis=1, keepdims=True)

    upd = m > best_ref[...]
    bestidx_ref[...] = jnp.where(upd, am, bestidx_ref[...])
    best_ref[...] = jnp.where(upd, m, best_ref[...])

    @pl.when(b == GRID - 1)
    def _done():
        out_ref[...] = bestidx_ref[...]


@functools.partial(jax.jit, static_argnames=())
def _sample(inputs):
    out = pl.pallas_call(
        _sample_kernel,
        grid=(GRID,),
        in_specs=[pl.BlockSpec((ROWS, BLK), lambda b: (0, b))],
        out_specs=pl.BlockSpec((ROWS, 1), lambda b: (0, 0)),
        out_shape=jax.ShapeDtypeStruct((ROWS, 1), jnp.int32),
        scratch_shapes=[
            pltpu.VMEM((ROWS, 1), jnp.float32),
            pltpu.VMEM((ROWS, 1), jnp.int32),
        ],
        compiler_params=pltpu.CompilerParams(
            dimension_semantics=("arbitrary",),
        ),
    )(inputs)
    return out.reshape(ROWS)


def kernel(inputs, states):
    predicted_ids = _sample(inputs)
    return (predicted_ids, states)


# speedup vs baseline: 1.4965x; 1.4965x over previous
import functools

import jax
import jax.numpy as jnp
from jax.experimental import pallas as pl
from jax.experimental.pallas import tpu as pltpu

ROWS = 32
VOCAB = 1_000_000
BLK = 8192
TW = 256
GRID = (VOCAB + BLK - 1) // BLK

_TINY = 1.1754943508222875e-38
_BIG_IDX = 2**30


def _threefry_bits(j):
    rotations = ((13, 15, 26, 6), (17, 29, 16, 24))
    k0 = jnp.uint32(0)
    k1 = jnp.uint32(42)
    ks = (k0, k1, jnp.uint32(0x1BD11BDA) ^ k0 ^ k1)
    x0 = jnp.zeros_like(j) + ks[0]
    x1 = j + ks[1]

    def rotl(x, d):
        return (x << jnp.uint32(d)) | (x >> jnp.uint32(32 - d))

    for i in range(5):
        for r in rotations[i % 2]:
            x0 = x0 + x1
            x1 = rotl(x1, r)
            x1 = x0 ^ x1
        x0 = x0 + ks[(i + 1) % 3]
        x1 = x1 + ks[(i + 2) % 3] + jnp.uint32(i + 1)
    return x0 ^ x1


def _partials_kernel(x_ref, vals_ref, idxs_ref):
    b = pl.program_id(0)
    row_off = jax.lax.broadcasted_iota(jnp.uint32, (ROWS, TW), 0) * jnp.uint32(VOCAB)
    lane = jax.lax.broadcasted_iota(jnp.uint32, (ROWS, TW), 1)
    tiny = jnp.float32(_TINY)

    def body(t, carry):
        acc_max, acc_idx = carry
        base = (b * BLK + t * TW).astype(jnp.uint32)
        col = lane + base
        j = row_off + col
        bits = _threefry_bits(j)
        ubits = (bits >> jnp.uint32(9)) | jnp.uint32(0x3F800000)
        f = jax.lax.bitcast_convert_type(ubits, jnp.float32) - jnp.float32(1.0)
        u = jnp.maximum(f + tiny, tiny)
        g = -jnp.log(-jnp.log(u))
        score = g + x_ref[:, pl.ds(t * TW, TW)]
        score = jnp.where(col < jnp.uint32(VOCAB), score, -jnp.inf)
        upd = score > acc_max
        acc_idx = jnp.where(upd, col.astype(jnp.int32), acc_idx)
        acc_max = jnp.maximum(acc_max, score)
        return acc_max, acc_idx

    acc_max0 = jnp.full((ROWS, TW), -jnp.inf, jnp.float32)
    acc_idx0 = jnp.zeros((ROWS, TW), jnp.int32)
    acc_max, acc_idx = jax.lax.fori_loop(0, BLK // TW, body, (acc_max0, acc_idx0))
    vals_ref[...] = acc_max
    idxs_ref[...] = acc_idx


def _merge_kernel(vals_ref, idxs_ref, out_ref):
    v = vals_ref[...]
    idx = idxs_ref[...]
    m = jnp.max(v, axis=1, keepdims=True)
    cand = jnp.where(v == m, idx, jnp.int32(_BIG_IDX))
    out_ref[...] = jnp.min(cand, axis=1, keepdims=True)


@jax.jit
def _sample(inputs):
    vals, idxs = pl.pallas_call(
        _partials_kernel,
        grid=(GRID,),
        in_specs=[pl.BlockSpec((ROWS, BLK), lambda b: (0, b))],
        out_specs=[
            pl.BlockSpec((ROWS, TW), lambda b: (0, b)),
            pl.BlockSpec((ROWS, TW), lambda b: (0, b)),
        ],
        out_shape=[
            jax.ShapeDtypeStruct((ROWS, GRID * TW), jnp.float32),
            jax.ShapeDtypeStruct((ROWS, GRID * TW), jnp.int32),
        ],
        compiler_params=pltpu.CompilerParams(
            dimension_semantics=("parallel",),
        ),
    )(inputs)
    out = pl.pallas_call(
        _merge_kernel,
        out_shape=jax.ShapeDtypeStruct((ROWS, 1), jnp.int32),
    )(vals, idxs)
    return out.reshape(ROWS)


def kernel(inputs, states):
    predicted_ids = _sample(inputs)
    return (predicted_ids, states)


# speedup vs baseline: 1.6493x; 1.1021x over previous
import functools

import jax
import jax.numpy as jnp
from jax.experimental import pallas as pl
from jax.experimental.pallas import tpu as pltpu

ROWS = 32
VOCAB = 1_000_000
BLK = 8192
TW = 128
U = 8
GRID = (VOCAB + BLK - 1) // BLK

_TINY = 1.1754943508222875e-38
_BIG_IDX = 2**30


def _threefry_bits(j):
    rotations = ((13, 15, 26, 6), (17, 29, 16, 24))
    k0 = jnp.uint32(0)
    k1 = jnp.uint32(42)
    ks = (k0, k1, jnp.uint32(0x1BD11BDA) ^ k0 ^ k1)
    x0 = jnp.zeros_like(j) + ks[0]
    x1 = j + ks[1]

    def rotl(x, d):
        return (x << jnp.uint32(d)) | (x >> jnp.uint32(32 - d))

    for i in range(5):
        for r in rotations[i % 2]:
            x0 = x0 + x1
            x1 = rotl(x1, r)
            x1 = x0 ^ x1
        x0 = x0 + ks[(i + 1) % 3]
        x1 = x1 + ks[(i + 2) % 3] + jnp.uint32(i + 1)
    return x0 ^ x1


def _partials_kernel(x_ref, vals_ref, idxs_ref):
    b = pl.program_id(0)
    row_off = jax.lax.broadcasted_iota(jnp.uint32, (ROWS, TW), 0) * jnp.uint32(VOCAB)
    lane = jax.lax.broadcasted_iota(jnp.uint32, (ROWS, TW), 1)
    tiny = jnp.float32(_TINY)

    def body(t, carry):
        acc_max, acc_idx = carry
        for s in range(U):
            base = (b * BLK + (t * U + s) * TW).astype(jnp.uint32)
            col = lane + base
            j = row_off + col
            bits = _threefry_bits(j)
            ubits = (bits >> jnp.uint32(9)) | jnp.uint32(0x3F800000)
            f = jax.lax.bitcast_convert_type(ubits, jnp.float32) - jnp.float32(1.0)
            u = jnp.maximum(f, tiny)
            g = -jnp.log(-jnp.log(u))
            score = g + x_ref[:, pl.ds((t * U + s) * TW, TW)]
            score = jnp.where(col < jnp.uint32(VOCAB), score, -jnp.inf)
            upd = score > acc_max
            acc_idx = jnp.where(upd, col.astype(jnp.int32), acc_idx)
            acc_max = jnp.maximum(acc_max, score)
        return acc_max, acc_idx

    acc_max0 = jnp.full((ROWS, TW), -jnp.inf, jnp.float32)
    acc_idx0 = jnp.zeros((ROWS, TW), jnp.int32)
    acc_max, acc_idx = jax.lax.fori_loop(0, BLK // (TW * U), body, (acc_max0, acc_idx0))
    vals_ref[...] = acc_max
    idxs_ref[...] = acc_idx


def _merge_kernel(vals_ref, idxs_ref, out_ref):
    v = vals_ref[...]
    idx = idxs_ref[...]
    m = jnp.max(v, axis=1, keepdims=True)
    cand = jnp.where(v == m, idx, jnp.int32(_BIG_IDX))
    out_ref[...] = jnp.min(cand, axis=1, keepdims=True)


@jax.jit
def _sample(inputs):
    vals, idxs = pl.pallas_call(
        _partials_kernel,
        grid=(GRID,),
        in_specs=[pl.BlockSpec((ROWS, BLK), lambda b: (0, b))],
        out_specs=[
            pl.BlockSpec((ROWS, TW), lambda b: (0, b)),
            pl.BlockSpec((ROWS, TW), lambda b: (0, b)),
        ],
        out_shape=[
            jax.ShapeDtypeStruct((ROWS, GRID * TW), jnp.float32),
            jax.ShapeDtypeStruct((ROWS, GRID * TW), jnp.int32),
        ],
        compiler_params=pltpu.CompilerParams(
            dimension_semantics=("parallel",),
        ),
    )(inputs)
    out = pl.pallas_call(
        _merge_kernel,
        out_shape=jax.ShapeDtypeStruct((ROWS, 1), jnp.int32),
    )(vals, idxs)
    return out.reshape(ROWS)


def kernel(inputs, states):
    predicted_ids = _sample(inputs)
    return (predicted_ids, states)


# speedup vs baseline: 1.6560x; 1.0040x over previous
import functools

import jax
import jax.numpy as jnp
from jax.experimental import pallas as pl
from jax.experimental.pallas import tpu as pltpu

ROWS = 32
VOCAB = 1_000_000
BLK = 16384
TW = 128
U = 16
GRID = (VOCAB + BLK - 1) // BLK

_TINY = 1.1754943508222875e-38
_BIG_IDX = 2**30


def _threefry_bits(j):
    rotations = ((13, 15, 26, 6), (17, 29, 16, 24))
    k0 = jnp.uint32(0)
    k1 = jnp.uint32(42)
    ks = (k0, k1, jnp.uint32(0x1BD11BDA) ^ k0 ^ k1)
    x0 = jnp.zeros_like(j) + ks[0]
    x1 = j + ks[1]

    def rotl(x, d):
        return (x << jnp.uint32(d)) | (x >> jnp.uint32(32 - d))

    for i in range(5):
        for r in rotations[i % 2]:
            x0 = x0 + x1
            x1 = rotl(x1, r)
            x1 = x0 ^ x1
        x0 = x0 + ks[(i + 1) % 3]
        x1 = x1 + ks[(i + 2) % 3] + jnp.uint32(i + 1)
    return x0 ^ x1


def _partials_kernel(x_ref, vals_ref, idxs_ref):
    b = pl.program_id(0)
    row_off = jax.lax.broadcasted_iota(jnp.uint32, (ROWS, TW), 0) * jnp.uint32(VOCAB)
    lane = jax.lax.broadcasted_iota(jnp.uint32, (ROWS, TW), 1)
    tiny = jnp.float32(_TINY)

    def body(t, carry):
        acc_max, acc_idx = carry
        for s in range(U):
            base = (b * BLK + (t * U + s) * TW).astype(jnp.uint32)
            col = lane + base
            j = row_off + col
            bits = _threefry_bits(j)
            ubits = (bits >> jnp.uint32(9)) | jnp.uint32(0x3F800000)
            f = jax.lax.bitcast_convert_type(ubits, jnp.float32) - jnp.float32(1.0)
            u = jnp.maximum(f, tiny)
            g = -jnp.log(-jnp.log(u))
            score = g + x_ref[:, pl.ds((t * U + s) * TW, TW)]
            score = jnp.where(col < jnp.uint32(VOCAB), score, -jnp.inf)
            upd = score > acc_max
            acc_idx = jnp.where(upd, col.astype(jnp.int32), acc_idx)
            acc_max = jnp.maximum(acc_max, score)
        return acc_max, acc_idx

    acc_max0 = jnp.full((ROWS, TW), -jnp.inf, jnp.float32)
    acc_idx0 = jnp.zeros((ROWS, TW), jnp.int32)
    acc_max, acc_idx = jax.lax.fori_loop(0, BLK // (TW * U), body, (acc_max0, acc_idx0))
    vals_ref[...] = acc_max
    idxs_ref[...] = acc_idx


def _merge_kernel(vals_ref, idxs_ref, out_ref):
    v = vals_ref[...]
    idx = idxs_ref[...]
    m = jnp.max(v, axis=1, keepdims=True)
    cand = jnp.where(v == m, idx, jnp.int32(_BIG_IDX))
    out_ref[...] = jnp.min(cand, axis=1, keepdims=True)


@jax.jit
def _sample(inputs):
    vals, idxs = pl.pallas_call(
        _partials_kernel,
        grid=(GRID,),
        in_specs=[pl.BlockSpec((ROWS, BLK), lambda b: (0, b))],
        out_specs=[
            pl.BlockSpec((ROWS, TW), lambda b: (0, b)),
            pl.BlockSpec((ROWS, TW), lambda b: (0, b)),
        ],
        out_shape=[
            jax.ShapeDtypeStruct((ROWS, GRID * TW), jnp.float32),
            jax.ShapeDtypeStruct((ROWS, GRID * TW), jnp.int32),
        ],
        compiler_params=pltpu.CompilerParams(
            dimension_semantics=("parallel",),
        ),
    )(inputs)
    out = pl.pallas_call(
        _merge_kernel,
        out_shape=jax.ShapeDtypeStruct((ROWS, 1), jnp.int32),
    )(vals, idxs)
    return out.reshape(ROWS)


def kernel(inputs, states):
    predicted_ids = _sample(inputs)
    return (predicted_ids, states)
